# Initial kernel scaffold; baseline (speedup 1.0000x reference)
import jax
import jax.numpy as jnp
from jax.experimental import pallas as pl


def kernel(data, W, gamma, beta):
    raise NotImplementedError("write your pallas kernel here")


# speedup vs baseline: 12.2103x; 12.2103x over previous
import functools

import jax
import jax.numpy as jnp
from jax import lax
from jax.experimental import pallas as pl
from jax.experimental.pallas import tpu as pltpu
from jax.experimental.pallas import tpu_sc as plsc

B, C, N, K = 4, 64, 4096, 20
C_OUT = 64
NPTS = B * N

RB = 256
NB = N // RB

_SCG = 4
_GK = _SCG * K
_NSC, _NSUB = 2, 16
_NW = _NSC * _NSUB
_PPW = NPTS // _NW

CH = 2048
G1 = NPTS // CH


def _topk_body(xf_ref, xr_ref, idx_ref):
    b = pl.program_id(0)
    x = xf_ref[0]
    xr = xr_ref[0]
    xx = jnp.sum(x * x, axis=0, keepdims=True)
    xxr = jnp.sum(xr * xr, axis=0)[:, None]
    g = lax.dot_general(xr.astype(jnp.bfloat16), x.astype(jnp.bfloat16),
                        (((0,), (0,)), ((), ())),
                        preferred_element_type=jnp.float32)
    inner = -2.0 * g
    d = -xx - inner - xxr
    iota = lax.broadcasted_iota(jnp.int32, (RB, N), 1)
    neg = jnp.float32(-jnp.inf)
    dw = d
    cols = []
    for _ in range(K):
        m = jnp.max(dw, axis=1, keepdims=True)
        j = jnp.min(jnp.where(dw == m, iota, N), axis=1, keepdims=True)
        cols.append(j)
        dw = jnp.where(iota == j, neg, dw)
    idx_ref[0] = jnp.concatenate(cols, axis=1) + b * N


def _proj_body(x_ref, w_ref, yz_ref, z_ref):
    x = x_ref[0]
    w = w_ref[...]
    yz = lax.dot_general(x, w, (((0,), (0,)), ((), ())),
                         preferred_element_type=jnp.float32,
                         precision=lax.Precision.HIGHEST)
    yz_ref[0] = yz
    z_ref[0] = yz[:, C_OUT:]


def _sc_body(y_hbm, idx_hbm, stats_hbm, ext_hbm,
             idx_v, rows_v, stats_v, ext_v, sem):
    wid = lax.axis_index("s") * _NSC + lax.axis_index("c")
    base = wid * _PPW

    def grp(g, carry):
        p0 = base + g * _SCG
        pltpu.sync_copy(idx_hbm.at[pl.ds(p0 * K, _GK)], idx_v)
        pltpu.async_copy(y_hbm.at[idx_v], rows_v, sem).wait()
        for p in range(_SCG):
            for c in range(4):
                sl = pl.ds(c * 16, 16)
                sh = pl.ds(C_OUT + c * 16, 16)
                v = rows_v[p * K, sl]
                s = v
                q = v * v
                mx = v
                mn = v
                for k in range(1, K):
                    v = rows_v[p * K + k, sl]
                    s = s + v
                    q = q + v * v
                    mx = jnp.maximum(mx, v)
                    mn = jnp.minimum(mn, v)
                stats_v[p, sl] = s
                stats_v[p, sh] = q
                ext_v[p, sl] = mx
                ext_v[p, sh] = mn
        pltpu.sync_copy(stats_v, stats_hbm.at[pl.ds(p0, _SCG)])
        pltpu.sync_copy(ext_v, ext_hbm.at[pl.ds(p0, _SCG)])
        return carry

    lax.fori_loop(0, _PPW // _SCG, grp, 0)


def _stats_body(r_ref, z_ref, o_ref):
    i = pl.program_id(0)
    sy = r_ref[:, :C_OUT]
    sq = r_ref[:, C_OUT:]
    z = z_ref[...]
    rows = jnp.concatenate([
        jnp.sum(sy, axis=0)[None],
        jnp.sum(sq, axis=0)[None],
        jnp.sum(z * sy, axis=0)[None],
        jnp.sum(z, axis=0)[None],
        jnp.sum(z * z, axis=0)[None],
        jnp.zeros((3, C_OUT), jnp.float32),
    ], axis=0)

    @pl.when(i == 0)
    def _():
        o_ref[...] = rows

    @pl.when(i > 0)
    def _():
        o_ref[...] = o_ref[...] + rows


def _final_body(e_ref, z_ref, st_ref, g_ref, b_ref, o_ref):
    st = st_ref[...]
    inv = jnp.float32(1.0 / (B * N * K))
    kf = jnp.float32(K)
    mean = (st[0:1] + kf * st[3:4]) * inv
    e2 = (st[1:2] + 2.0 * st[2:3] + kf * st[4:5]) * inv
    var = e2 - mean * mean
    istd = lax.rsqrt(var + 1e-5)
    scale = g_ref[...] * istd
    shift = b_ref[...] - mean * scale
    mx = e_ref[:, :C_OUT]
    mn = e_ref[:, C_OUT:]
    v = jnp.where(scale >= 0.0, mx, mn) + z_ref[...]
    t = v * scale + shift
    o_ref[...] = jnp.where(t > 0.0, t, 0.2 * t)


def _topk_call(data):
    return pl.pallas_call(
        _topk_body,
        grid=(B, NB),
        in_specs=[
            pl.BlockSpec((1, C, N), lambda b, i: (b, 0, 0)),
            pl.BlockSpec((1, C, RB), lambda b, i: (b, 0, i)),
        ],
        out_specs=pl.BlockSpec((1, RB, K), lambda b, i: (b, i, 0)),
        out_shape=jax.ShapeDtypeStruct((B, N, K), jnp.int32),
    )(data, data)


def _proj_call(data, wcat):
    return pl.pallas_call(
        _proj_body,
        grid=(B,),
        in_specs=[
            pl.BlockSpec((1, C, N), lambda b: (b, 0, 0)),
            pl.BlockSpec((C, 2 * C_OUT), lambda b: (0, 0)),
        ],
        out_specs=[
            pl.BlockSpec((1, N, 2 * C_OUT), lambda b: (b, 0, 0)),
            pl.BlockSpec((1, N, C_OUT), lambda b: (b, 0, 0)),
        ],
        out_shape=[
            jax.ShapeDtypeStruct((B, N, 2 * C_OUT), jnp.float32),
            jax.ShapeDtypeStruct((B, N, C_OUT), jnp.float32),
        ],
    )(data, wcat)


@functools.cache
def _sc_kernel():
    return pl.kernel(
        _sc_body,
        out_type=(
            jax.ShapeDtypeStruct((NPTS, 2 * C_OUT), jnp.float32),
            jax.ShapeDtypeStruct((NPTS, 2 * C_OUT), jnp.float32),
        ),
        mesh=plsc.VectorSubcoreMesh(core_axis_name="c", subcore_axis_name="s"),
        scratch_types=[
            pltpu.VMEM((_GK,), jnp.int32),
            pltpu.VMEM((_GK, 2 * C_OUT), jnp.float32),
            pltpu.VMEM((_SCG, 2 * C_OUT), jnp.float32),
            pltpu.VMEM((_SCG, 2 * C_OUT), jnp.float32),
            pltpu.SemaphoreType.DMA,
        ],
    )


def _sc_call(y2, idxflat):
    return _sc_kernel()(y2, idxflat)


def _stats_call(rstats, z2):
    return pl.pallas_call(
        _stats_body,
        grid=(G1,),
        in_specs=[
            pl.BlockSpec((CH, 2 * C_OUT), lambda i: (i, 0)),
            pl.BlockSpec((CH, C_OUT), lambda i: (i, 0)),
        ],
        out_specs=pl.BlockSpec((8, C_OUT), lambda i: (0, 0)),
        out_shape=jax.ShapeDtypeStruct((8, C_OUT), jnp.float32),
    )(rstats, z2)


def _final_call(rext, z2, st, gamma2, beta2):
    return pl.pallas_call(
        _final_body,
        grid=(G1,),
        in_specs=[
            pl.BlockSpec((CH, 2 * C_OUT), lambda i: (i, 0)),
            pl.BlockSpec((CH, C_OUT), lambda i: (i, 0)),
            pl.BlockSpec((8, C_OUT), lambda i: (0, 0)),
            pl.BlockSpec((1, C_OUT), lambda i: (0, 0)),
            pl.BlockSpec((1, C_OUT), lambda i: (0, 0)),
        ],
        out_specs=pl.BlockSpec((CH, C_OUT), lambda i: (i, 0)),
        out_shape=jax.ShapeDtypeStruct((NPTS, C_OUT), jnp.float32),
    )(rext, z2, st, gamma2, beta2)


def kernel(data, W, gamma, beta):
    w1 = W[:, :C]
    w2 = W[:, C:]
    wcat = jnp.concatenate([w1.T, (w2 - w1).T], axis=1)
    idx = _topk_call(data)
    yzt, zt = _proj_call(data, wcat)
    y2 = yzt.reshape(NPTS, 2 * C_OUT)
    z2 = zt.reshape(NPTS, C_OUT)
    stats, ext = _sc_call(y2, idx.reshape(NPTS * K))
    st = _stats_call(stats, z2)
    outp = _final_call(ext, z2, st,
                       gamma.reshape(1, C_OUT), beta.reshape(1, C_OUT))
    return outp.reshape(B, N, C_OUT).transpose(0, 2, 1)


# speedup vs baseline: 13.8109x; 1.1311x over previous
import functools

import jax
import jax.numpy as jnp
from jax import lax
from jax.experimental import pallas as pl
from jax.experimental.pallas import tpu as pltpu
from jax.experimental.pallas import tpu_sc as plsc

B, C, N, K = 4, 64, 4096, 20
C_OUT = 64
NPTS = B * N

RB = 256
NB = N // RB

_SCG = 4
_GK = _SCG * K
_NSC, _NSUB = 2, 16
_NW = _NSC * _NSUB
_PPW = NPTS // _NW

CH = 2048
G1 = NPTS // CH


def _topk_body(xf_ref, xr_ref, idx_ref):
    b = pl.program_id(0)
    x = xf_ref[0]
    xr = xr_ref[0]
    xx = jnp.sum(x * x, axis=0, keepdims=True)
    xxr = jnp.sum(xr * xr, axis=0)[:, None]
    g = lax.dot_general(xr.astype(jnp.bfloat16), x.astype(jnp.bfloat16),
                        (((0,), (0,)), ((), ())),
                        preferred_element_type=jnp.float32)
    inner = -2.0 * g
    d = -xx - inner - xxr
    iota = lax.broadcasted_iota(jnp.int32, (RB, N), 1)
    neg = jnp.float32(-jnp.inf)
    dw = d
    cols = []
    for _ in range(K):
        j = jnp.argmax(dw, axis=1).astype(jnp.int32)[:, None]
        cols.append(j)
        dw = jnp.where(iota == j, neg, dw)
    idx_ref[0] = jnp.concatenate(cols, axis=1) + b * N


def _proj_body(x_ref, w_ref, yz_ref, z_ref):
    x = x_ref[0]
    w = w_ref[...]
    yz = lax.dot_general(x, w, (((0,), (0,)), ((), ())),
                         preferred_element_type=jnp.float32,
                         precision=lax.Precision.HIGHEST)
    yz_ref[0] = yz
    z_ref[0] = yz[:, C_OUT:]


def _sc_body(y_hbm, idx_hbm, stats_hbm, ext_hbm,
             idx_v, rows_v, stats_v, ext_v, sem):
    wid = lax.axis_index("s") * _NSC + lax.axis_index("c")
    base = wid * _PPW

    def grp(g, carry):
        p0 = base + g * _SCG
        pltpu.sync_copy(idx_hbm.at[pl.ds(p0 * K, _GK)], idx_v)
        pltpu.async_copy(y_hbm.at[idx_v], rows_v, sem).wait()
        for p in range(_SCG):
            for c in range(4):
                sl = pl.ds(c * 16, 16)
                sh = pl.ds(C_OUT + c * 16, 16)
                v = rows_v[p * K, sl]
                s = v
                q = v * v
                mx = v
                mn = v
                for k in range(1, K):
                    v = rows_v[p * K + k, sl]
                    s = s + v
                    q = q + v * v
                    mx = jnp.maximum(mx, v)
                    mn = jnp.minimum(mn, v)
                stats_v[p, sl] = s
                stats_v[p, sh] = q
                ext_v[p, sl] = mx
                ext_v[p, sh] = mn
        pltpu.sync_copy(stats_v, stats_hbm.at[pl.ds(p0, _SCG)])
        pltpu.sync_copy(ext_v, ext_hbm.at[pl.ds(p0, _SCG)])
        return carry

    lax.fori_loop(0, _PPW // _SCG, grp, 0)


def _stats_body(r_ref, z_ref, o_ref):
    i = pl.program_id(0)
    sy = r_ref[:, :C_OUT]
    sq = r_ref[:, C_OUT:]
    z = z_ref[...]
    rows = jnp.concatenate([
        jnp.sum(sy, axis=0)[None],
        jnp.sum(sq, axis=0)[None],
        jnp.sum(z * sy, axis=0)[None],
        jnp.sum(z, axis=0)[None],
        jnp.sum(z * z, axis=0)[None],
        jnp.zeros((3, C_OUT), jnp.float32),
    ], axis=0)

    @pl.when(i == 0)
    def _():
        o_ref[...] = rows

    @pl.when(i > 0)
    def _():
        o_ref[...] = o_ref[...] + rows


def _final_body(e_ref, z_ref, st_ref, g_ref, b_ref, o_ref):
    st = st_ref[...]
    inv = jnp.float32(1.0 / (B * N * K))
    kf = jnp.float32(K)
    mean = (st[0:1] + kf * st[3:4]) * inv
    e2 = (st[1:2] + 2.0 * st[2:3] + kf * st[4:5]) * inv
    var = e2 - mean * mean
    istd = lax.rsqrt(var + 1e-5)
    scale = g_ref[...] * istd
    shift = b_ref[...] - mean * scale
    mx = e_ref[:, :C_OUT]
    mn = e_ref[:, C_OUT:]
    v = jnp.where(scale >= 0.0, mx, mn) + z_ref[...]
    t = v * scale + shift
    o_ref[...] = jnp.where(t > 0.0, t, 0.2 * t)


def _topk_call(data):
    return pl.pallas_call(
        _topk_body,
        grid=(B, NB),
        in_specs=[
            pl.BlockSpec((1, C, N), lambda b, i: (b, 0, 0)),
            pl.BlockSpec((1, C, RB), lambda b, i: (b, 0, i)),
        ],
        out_specs=pl.BlockSpec((1, RB, K), lambda b, i: (b, i, 0)),
        out_shape=jax.ShapeDtypeStruct((B, N, K), jnp.int32),
    )(data, data)


def _proj_call(data, wcat):
    return pl.pallas_call(
        _proj_body,
        grid=(B,),
        in_specs=[
            pl.BlockSpec((1, C, N), lambda b: (b, 0, 0)),
            pl.BlockSpec((C, 2 * C_OUT), lambda b: (0, 0)),
        ],
        out_specs=[
            pl.BlockSpec((1, N, 2 * C_OUT), lambda b: (b, 0, 0)),
            pl.BlockSpec((1, N, C_OUT), lambda b: (b, 0, 0)),
        ],
        out_shape=[
            jax.ShapeDtypeStruct((B, N, 2 * C_OUT), jnp.float32),
            jax.ShapeDtypeStruct((B, N, C_OUT), jnp.float32),
        ],
    )(data, wcat)


@functools.cache
def _sc_kernel():
    return pl.kernel(
        _sc_body,
        out_type=(
            jax.ShapeDtypeStruct((NPTS, 2 * C_OUT), jnp.float32),
            jax.ShapeDtypeStruct((NPTS, 2 * C_OUT), jnp.float32),
        ),
        mesh=plsc.VectorSubcoreMesh(core_axis_name="c", subcore_axis_name="s"),
        scratch_types=[
            pltpu.VMEM((_GK,), jnp.int32),
            pltpu.VMEM((_GK, 2 * C_OUT), jnp.float32),
            pltpu.VMEM((_SCG, 2 * C_OUT), jnp.float32),
            pltpu.VMEM((_SCG, 2 * C_OUT), jnp.float32),
            pltpu.SemaphoreType.DMA,
        ],
    )


def _sc_call(y2, idxflat):
    return _sc_kernel()(y2, idxflat)


def _stats_call(rstats, z2):
    return pl.pallas_call(
        _stats_body,
        grid=(G1,),
        in_specs=[
            pl.BlockSpec((CH, 2 * C_OUT), lambda i: (i, 0)),
            pl.BlockSpec((CH, C_OUT), lambda i: (i, 0)),
        ],
        out_specs=pl.BlockSpec((8, C_OUT), lambda i: (0, 0)),
        out_shape=jax.ShapeDtypeStruct((8, C_OUT), jnp.float32),
    )(rstats, z2)


def _final_call(rext, z2, st, gamma2, beta2):
    return pl.pallas_call(
        _final_body,
        grid=(G1,),
        in_specs=[
            pl.BlockSpec((CH, 2 * C_OUT), lambda i: (i, 0)),
            pl.BlockSpec((CH, C_OUT), lambda i: (i, 0)),
            pl.BlockSpec((8, C_OUT), lambda i: (0, 0)),
            pl.BlockSpec((1, C_OUT), lambda i: (0, 0)),
            pl.BlockSpec((1, C_OUT), lambda i: (0, 0)),
        ],
        out_specs=pl.BlockSpec((CH, C_OUT), lambda i: (i, 0)),
        out_shape=jax.ShapeDtypeStruct((NPTS, C_OUT), jnp.float32),
    )(rext, z2, st, gamma2, beta2)


def kernel(data, W, gamma, beta):
    w1 = W[:, :C]
    w2 = W[:, C:]
    wcat = jnp.concatenate([w1.T, (w2 - w1).T], axis=1)
    idx = _topk_call(data)
    yzt, zt = _proj_call(data, wcat)
    y2 = yzt.reshape(NPTS, 2 * C_OUT)
    z2 = zt.reshape(NPTS, C_OUT)
    stats, ext = _sc_call(y2, idx.reshape(NPTS * K))
    st = _stats_call(stats, z2)
    outp = _final_call(ext, z2, st,
                       gamma.reshape(1, C_OUT), beta.reshape(1, C_OUT))
    return outp.reshape(B, N, C_OUT).transpose(0, 2, 1)


# speedup vs baseline: 14.7177x; 1.0657x over previous
import functools

import jax
import jax.numpy as jnp
from jax import lax
from jax.experimental import pallas as pl
from jax.experimental.pallas import tpu as pltpu
from jax.experimental.pallas import tpu_sc as plsc

B, C, N, K = 4, 64, 4096, 20
C_OUT = 64
NPTS = B * N

RB = 256
NB = N // RB

_SCG = 4
_GK = _SCG * K
_NSC, _NSUB = 2, 16
_NW = _NSC * _NSUB
_PPW = NPTS // _NW

CH = 2048
G1 = NPTS // CH


def _topk_body(xf_ref, xr_ref, idx_ref):
    b = pl.program_id(0)
    x = xf_ref[0]
    xr = xr_ref[0]
    xx = jnp.sum(x * x, axis=0, keepdims=True)
    xxr = jnp.sum(xr * xr, axis=0)[:, None]
    g = lax.dot_general(xr.astype(jnp.bfloat16), x.astype(jnp.bfloat16),
                        (((0,), (0,)), ((), ())),
                        preferred_element_type=jnp.float32)
    inner = -2.0 * g
    d = -xx - inner - xxr
    iota = lax.broadcasted_iota(jnp.int32, (RB, N), 1)
    neg = jnp.float32(-jnp.inf)
    dw = d
    cols = []
    for _ in range(K):
        j = jnp.argmax(dw, axis=1).astype(jnp.int32)[:, None]
        cols.append(j)
        dw = jnp.where(iota == j, neg, dw)
    idx_ref[0] = jnp.concatenate(cols, axis=1) + b * N


def _proj_body(x_ref, w_ref, yz_ref, z_ref):
    x = x_ref[0]
    w = w_ref[...]
    yz = lax.dot_general(x, w, (((0,), (0,)), ((), ())),
                         preferred_element_type=jnp.float32,
                         precision=lax.Precision.HIGHEST)
    yz_ref[0] = yz
    z_ref[0] = yz[:, C_OUT:]


_NGRP = _PPW // _SCG
_SEGP = 128
_SEGG = _SEGP // _SCG


def _sc_body(y_hbm, idx_hbm, stats_hbm, ext_hbm,
             idx_v, rows0, rows1, stats_seg, ext_seg, sem0, sem1):
    wid = lax.axis_index("s") * _NSC + lax.axis_index("c")
    base = wid * _PPW

    pltpu.sync_copy(idx_hbm.at[pl.ds(base * K, _PPW * K)], idx_v)

    def gather(gl, rows, sem):
        pltpu.async_copy(y_hbm.at[idx_v.at[pl.ds(gl * _GK, _GK)]], rows, sem)

    def reduce_group(gl, rows):
        srow = (gl % _SEGG) * _SCG
        for p in range(_SCG):
            for c in range(4):
                sl = pl.ds(c * 16, 16)
                sh = pl.ds(C_OUT + c * 16, 16)
                v = rows[p * K, sl]
                s = v
                q = v * v
                mx = v
                mn = v
                for k in range(1, K):
                    v = rows[p * K + k, sl]
                    s = s + v
                    q = q + v * v
                    mx = jnp.maximum(mx, v)
                    mn = jnp.minimum(mn, v)
                stats_seg[srow + p, sl] = s
                stats_seg[srow + p, sh] = q
                ext_seg[srow + p, sl] = mx
                ext_seg[srow + p, sh] = mn

    gather(0, rows0, sem0)
    gather(1, rows1, sem1)

    def pair(g2, carry):
        def drain(rows, sem):
            pltpu.make_async_copy(y_hbm.at[idx_v.at[pl.ds(0, _GK)]],
                                  rows, sem).wait()

        drain(rows0, sem0)
        reduce_group(2 * g2, rows0)

        @pl.when(g2 < _NGRP // 2 - 1)
        def _():
            gather(2 * g2 + 2, rows0, sem0)

        drain(rows1, sem1)
        reduce_group(2 * g2 + 1, rows1)

        @pl.when(g2 < _NGRP // 2 - 1)
        def _():
            gather(2 * g2 + 3, rows1, sem1)

        @pl.when((g2 + 1) % (_SEGG // 2) == 0)
        def _():
            p0 = base + (g2 // (_SEGG // 2)) * _SEGP
            pltpu.sync_copy(stats_seg, stats_hbm.at[pl.ds(p0, _SEGP)])
            pltpu.sync_copy(ext_seg, ext_hbm.at[pl.ds(p0, _SEGP)])

        return carry

    lax.fori_loop(0, _NGRP // 2, pair, 0)


def _stats_body(r_ref, z_ref, o_ref):
    i = pl.program_id(0)
    sy = r_ref[:, :C_OUT]
    sq = r_ref[:, C_OUT:]
    z = z_ref[...]
    rows = jnp.concatenate([
        jnp.sum(sy, axis=0)[None],
        jnp.sum(sq, axis=0)[None],
        jnp.sum(z * sy, axis=0)[None],
        jnp.sum(z, axis=0)[None],
        jnp.sum(z * z, axis=0)[None],
        jnp.zeros((3, C_OUT), jnp.float32),
    ], axis=0)

    @pl.when(i == 0)
    def _():
        o_ref[...] = rows

    @pl.when(i > 0)
    def _():
        o_ref[...] = o_ref[...] + rows


def _final_body(e_ref, z_ref, st_ref, g_ref, b_ref, o_ref):
    st = st_ref[...]
    inv = jnp.float32(1.0 / (B * N * K))
    kf = jnp.float32(K)
    mean = (st[0:1] + kf * st[3:4]) * inv
    e2 = (st[1:2] + 2.0 * st[2:3] + kf * st[4:5]) * inv
    var = e2 - mean * mean
    istd = lax.rsqrt(var + 1e-5)
    scale = g_ref[...] * istd
    shift = b_ref[...] - mean * scale
    mx = e_ref[:, :C_OUT]
    mn = e_ref[:, C_OUT:]
    v = jnp.where(scale >= 0.0, mx, mn) + z_ref[...]
    t = v * scale + shift
    o_ref[...] = jnp.where(t > 0.0, t, 0.2 * t)


def _topk_call(data):
    return pl.pallas_call(
        _topk_body,
        grid=(B, NB),
        in_specs=[
            pl.BlockSpec((1, C, N), lambda b, i: (b, 0, 0)),
            pl.BlockSpec((1, C, RB), lambda b, i: (b, 0, i)),
        ],
        out_specs=pl.BlockSpec((1, RB, K), lambda b, i: (b, i, 0)),
        out_shape=jax.ShapeDtypeStruct((B, N, K), jnp.int32),
    )(data, data)


def _proj_call(data, wcat):
    return pl.pallas_call(
        _proj_body,
        grid=(B,),
        in_specs=[
            pl.BlockSpec((1, C, N), lambda b: (b, 0, 0)),
            pl.BlockSpec((C, 2 * C_OUT), lambda b: (0, 0)),
        ],
        out_specs=[
            pl.BlockSpec((1, N, 2 * C_OUT), lambda b: (b, 0, 0)),
            pl.BlockSpec((1, N, C_OUT), lambda b: (b, 0, 0)),
        ],
        out_shape=[
            jax.ShapeDtypeStruct((B, N, 2 * C_OUT), jnp.float32),
            jax.ShapeDtypeStruct((B, N, C_OUT), jnp.float32),
        ],
    )(data, wcat)


@functools.cache
def _sc_kernel():
    return pl.kernel(
        _sc_body,
        out_type=(
            jax.ShapeDtypeStruct((NPTS, 2 * C_OUT), jnp.float32),
            jax.ShapeDtypeStruct((NPTS, 2 * C_OUT), jnp.float32),
        ),
        mesh=plsc.VectorSubcoreMesh(core_axis_name="c", subcore_axis_name="s"),
        scratch_types=[
            pltpu.VMEM((_PPW * K,), jnp.int32),
            pltpu.VMEM((_GK, 2 * C_OUT), jnp.float32),
            pltpu.VMEM((_GK, 2 * C_OUT), jnp.float32),
            pltpu.VMEM((_SEGP, 2 * C_OUT), jnp.float32),
            pltpu.VMEM((_SEGP, 2 * C_OUT), jnp.float32),
            pltpu.SemaphoreType.DMA,
            pltpu.SemaphoreType.DMA,
        ],
    )


def _sc_call(y2, idxflat):
    return _sc_kernel()(y2, idxflat)


def _stats_call(rstats, z2):
    return pl.pallas_call(
        _stats_body,
        grid=(G1,),
        in_specs=[
            pl.BlockSpec((CH, 2 * C_OUT), lambda i: (i, 0)),
            pl.BlockSpec((CH, C_OUT), lambda i: (i, 0)),
        ],
        out_specs=pl.BlockSpec((8, C_OUT), lambda i: (0, 0)),
        out_shape=jax.ShapeDtypeStruct((8, C_OUT), jnp.float32),
    )(rstats, z2)


def _final_call(rext, z2, st, gamma2, beta2):
    return pl.pallas_call(
        _final_body,
        grid=(G1,),
        in_specs=[
            pl.BlockSpec((CH, 2 * C_OUT), lambda i: (i, 0)),
            pl.BlockSpec((CH, C_OUT), lambda i: (i, 0)),
            pl.BlockSpec((8, C_OUT), lambda i: (0, 0)),
            pl.BlockSpec((1, C_OUT), lambda i: (0, 0)),
            pl.BlockSpec((1, C_OUT), lambda i: (0, 0)),
        ],
        out_specs=pl.BlockSpec((CH, C_OUT), lambda i: (i, 0)),
        out_shape=jax.ShapeDtypeStruct((NPTS, C_OUT), jnp.float32),
    )(rext, z2, st, gamma2, beta2)


def kernel(data, W, gamma, beta):
    w1 = W[:, :C]
    w2 = W[:, C:]
    wcat = jnp.concatenate([w1.T, (w2 - w1).T], axis=1)
    idx = _topk_call(data)
    yzt, zt = _proj_call(data, wcat)
    y2 = yzt.reshape(NPTS, 2 * C_OUT)
    z2 = zt.reshape(NPTS, C_OUT)
    stats, ext = _sc_call(y2, idx.reshape(NPTS * K))
    st = _stats_call(stats, z2)
    outp = _final_call(ext, z2, st,
                       gamma.reshape(1, C_OUT), beta.reshape(1, C_OUT))
    return outp.reshape(B, N, C_OUT).transpose(0, 2, 1)
